# baseline (speedup 1.0000x reference)
import functools

import jax
import jax.numpy as jnp
from jax.experimental import pallas as pl
from jax.experimental.pallas import tpu as pltpu

_SR = 16000
_WT_LEN = 512
_N_WT = 10
_INC_SCALE = _WT_LEN / _SR

_LP = 65536
_C = _LP // 128
_RR = _C // 8


def _synth_body(p_ref, p0_ref, e_ref, att_ref, a_ref, d_ref, o_ref, m_ref):
    inc = p_ref[0] * _INC_SCALE
    inc0 = p0_ref[0] * _INC_SCALE

    lane = jax.lax.broadcasted_iota(jnp.int32, (_C, 128), 1)
    x = inc
    for d in (1, 2, 4, 8, 16, 32, 64):
        x = x + jnp.where(lane >= d, jnp.roll(x, d, axis=1), 0.0)

    rowtot = x[:, 127:128]
    row = jax.lax.broadcasted_iota(jnp.int32, (_C, 1), 0)
    s = rowtot
    for d in (1, 2, 4, 8, 16, 32, 64, 128, 256):
        s = s + jnp.where(row >= d, jnp.roll(s, d, axis=0), 0.0)
    cum = x + (s - rowtot)

    raw = cum - inc0
    m = raw - jnp.floor(raw * (1.0 / _WT_LEN)) * _WT_LEN
    m = jnp.where(_WT_LEN - m < 1e-5, 0.0, m)
    m_ref[...] = m

    a_full = a_ref[...]
    d_full = d_ref[...]

    def chunk(rr, carry):
        base = pl.multiple_of(rr * 8, 8)
        m8 = m_ref[pl.ds(base, 8), :]
        e8 = e_ref[0, pl.ds(base, 8), :]
        lbase = pl.multiple_of(rr * 1024, 1024)
        att8 = att_ref[0, :, pl.ds(lbase, 1024)]
        rows = []
        for q in range(8):
            mq = m8[q:q + 1, :]
            lowf = jnp.floor(mq)
            alpha = mq - lowf
            low = jnp.broadcast_to(lowf.astype(jnp.int32), (_N_WT, 128))
            av = jnp.zeros((_N_WT, 128), jnp.float32)
            dv = jnp.zeros((_N_WT, 128), jnp.float32)
            for c in range(4):
                rel = low - (128 * c)
                ok = (rel >= 0) & (rel < 128)
                cl = jnp.clip(rel, 0, 127)
                ga = jnp.take_along_axis(a_full[:, 128 * c:128 * (c + 1)], cl, axis=1)
                gd = jnp.take_along_axis(d_full[:, 128 * c:128 * (c + 1)], cl, axis=1)
                av = jnp.where(ok, ga, av)
                dv = jnp.where(ok, gd, dv)
            val = av + alpha * dv
            attq = att8[:, 128 * q:128 * (q + 1)]
            rows.append(jnp.sum(val * attq, axis=0, keepdims=True))
        o_ref[0, pl.ds(base, 8), :] = jnp.concatenate(rows, axis=0) * e8
        return carry

    jax.lax.fori_loop(0, _RR, chunk, 0)


def kernel(pitch, envelope, attention, wavetables):
    n, l = pitch.shape[0], pitch.shape[1]
    pad = _LP - l

    p2 = jnp.pad(pitch[..., 0], ((0, 0), (0, pad))).reshape(n, _C, 128)
    e2 = jnp.pad(envelope[..., 0], ((0, 0), (0, pad))).reshape(n, _C, 128)
    att3 = jnp.pad(attention.transpose(0, 2, 1), ((0, 0), (0, 0), (0, pad)))

    a_tab = wavetables
    d_tab = jnp.roll(wavetables, -1, axis=1) - wavetables

    out = pl.pallas_call(
        _synth_body,
        grid=(n,),
        in_specs=[
            pl.BlockSpec((1, _C, 128), lambda i: (i, 0, 0)),
            pl.BlockSpec((1, _C, 128), lambda i: (0, 0, 0)),
            pl.BlockSpec((1, _C, 128), lambda i: (i, 0, 0)),
            pl.BlockSpec((1, _N_WT, _LP), lambda i: (i, 0, 0)),
            pl.BlockSpec((_N_WT, _WT_LEN), lambda i: (0, 0)),
            pl.BlockSpec((_N_WT, _WT_LEN), lambda i: (0, 0)),
        ],
        out_specs=pl.BlockSpec((1, _C, 128), lambda i: (i, 0, 0)),
        out_shape=jax.ShapeDtypeStruct((n, _C, 128), jnp.float32),
        scratch_shapes=[pltpu.VMEM((_C, 128), jnp.float32)],
        compiler_params=pltpu.CompilerParams(
            dimension_semantics=("parallel",)),
    )(p2, p2, e2, att3, a_tab, d_tab)

    return out.reshape(n, _LP)[:, :l, None]


# speedup vs baseline: 1.6351x; 1.6351x over previous
import jax
import jax.numpy as jnp
from jax.experimental import pallas as pl
from jax.experimental.pallas import tpu as pltpu

_SR = 16000
_WT_LEN = 512
_N_WT = 10
_INC_SCALE = _WT_LEN / _SR

_LP = 65536
_C = _LP // 128
_RR = _C // 8


def _synth_body(b64_ref, p_ref, p0_ref, e_ref, att_ref, a_ref, d_ref,
                o_ref, m_ref):
    n = pl.program_id(0)

    inc = p_ref[0] * _INC_SCALE
    inc0 = p0_ref[0] * _INC_SCALE

    lane = jax.lax.broadcasted_iota(jnp.int32, (_C, 128), 1)
    x = inc
    for d in (1, 2, 4, 8, 16, 32, 64):
        x = x + jnp.where(lane >= d, jnp.roll(x, d, axis=1), 0.0)

    rowtot = x[:, 127:128]
    row = jax.lax.broadcasted_iota(jnp.int32, (_C, 1), 0)
    s = rowtot
    for d in (1, 2, 4, 8, 16, 32, 64, 128, 256):
        s = s + jnp.where(row >= d, jnp.roll(s, d, axis=0), 0.0)
    cum = x + (s - rowtot)

    raw = cum - inc0
    m = raw - jnp.floor(raw * (1.0 / _WT_LEN)) * _WT_LEN
    m_ref[...] = m

    def chunk(rr, carry):
        base = pl.multiple_of(rr * 8, 8)
        m8 = m_ref[pl.ds(base, 8), :]
        e8 = e_ref[0, pl.ds(base, 8), :]
        b64 = b64_ref[n, rr]
        woff = pl.multiple_of(b64 * 16, 8)
        a_win = a_ref[pl.ds(woff, 16), :]
        d_win = d_ref[pl.ds(woff, 16), :]
        shift = b64 * 64
        lbase = pl.multiple_of(rr * 1024, 1024)
        rows = []
        for q in range(8):
            mq = m8[q:q + 1, :]
            lowf = jnp.floor(mq)
            alpha = mq - lowf
            rel = ((lowf.astype(jnp.int32) - shift) & (_WT_LEN - 1)) & 127
            relb = jnp.broadcast_to(rel, (16, 128))
            av = jnp.take_along_axis(a_win, relb, axis=1)[:_N_WT]
            dv = jnp.take_along_axis(d_win, relb, axis=1)[:_N_WT]
            val = av + alpha * dv
            attq = att_ref[0, :, pl.ds(lbase + 128 * q, 128)]
            rows.append(jnp.sum(val * attq, axis=0, keepdims=True))
        o_ref[0, pl.ds(base, 8), :] = jnp.concatenate(rows, axis=0) * e8
        return carry

    jax.lax.fori_loop(0, _RR, chunk, 0)


def kernel(pitch, envelope, attention, wavetables):
    n, l = pitch.shape[0], pitch.shape[1]
    pad = _LP - l

    p2 = jnp.pad(pitch[..., 0], ((0, 0), (0, pad))).reshape(n, _C, 128)
    e2 = jnp.pad(envelope[..., 0], ((0, 0), (0, pad))).reshape(n, _C, 128)
    att3 = jnp.pad(attention.transpose(0, 2, 1), ((0, 0), (0, 0), (0, pad)))

    a_tab = wavetables
    d_tab = jnp.roll(wavetables, -1, axis=1) - wavetables

    def windows(t):
        t2 = jnp.concatenate([t, t], axis=1)
        w = jnp.stack([t2[:, 64 * k:64 * k + 128] for k in range(8)])
        return jnp.pad(w, ((0, 0), (0, 16 - _N_WT), (0, 0))).reshape(128, 128)

    a8, d8 = windows(a_tab), windows(d_tab)

    incp = jnp.pad(pitch[..., 0], ((0, 0), (0, pad))) * _INC_SCALE
    blk = incp.reshape(n, _RR, 1024)
    excl = jnp.cumsum(blk.sum(axis=2), axis=1) - blk.sum(axis=2)
    raw_first = excl + blk[:, :, 0] - blk[0, :, 0]
    basef = raw_first - 1.0
    basef = basef - jnp.floor(basef * (1.0 / _WT_LEN)) * _WT_LEN
    b64 = ((basef.astype(jnp.int32) & (_WT_LEN - 1)) >> 6)

    out = pl.pallas_call(
        _synth_body,
        grid_spec=pltpu.PrefetchScalarGridSpec(
            num_scalar_prefetch=1,
            grid=(n,),
            in_specs=[
                pl.BlockSpec((1, _C, 128), lambda i, b: (i, 0, 0)),
                pl.BlockSpec((1, _C, 128), lambda i, b: (0, 0, 0)),
                pl.BlockSpec((1, _C, 128), lambda i, b: (i, 0, 0)),
                pl.BlockSpec((1, _N_WT, _LP), lambda i, b: (i, 0, 0)),
                pl.BlockSpec((128, 128), lambda i, b: (0, 0)),
                pl.BlockSpec((128, 128), lambda i, b: (0, 0)),
            ],
            out_specs=pl.BlockSpec((1, _C, 128), lambda i, b: (i, 0, 0)),
            scratch_shapes=[pltpu.VMEM((_C, 128), jnp.float32)],
        ),
        out_shape=jax.ShapeDtypeStruct((n, _C, 128), jnp.float32),
        compiler_params=pltpu.CompilerParams(
            dimension_semantics=("parallel",)),
    )(b64, p2, p2, e2, att3, a8, d8)

    return out.reshape(n, _LP)[:, :l, None]


# speedup vs baseline: 2.3625x; 1.4448x over previous
import jax
import jax.numpy as jnp
from jax.experimental import pallas as pl
from jax.experimental.pallas import tpu as pltpu

_SR = 16000
_WT_LEN = 512
_N_WT = 10
_INC_SCALE = _WT_LEN / _SR

_LP = 65536
_C = _LP // 128
_RR = _C // 8


def _synth_body(b64_ref, p_ref, p0_ref, e_ref, att_ref, a_ref, d_ref,
                o_ref, m_ref):
    n = pl.program_id(0)

    inc = p_ref[0] * _INC_SCALE
    inc0 = p0_ref[0] * _INC_SCALE

    lane = jax.lax.broadcasted_iota(jnp.int32, (_C, 128), 1)
    x = inc
    for d in (1, 2, 4, 8, 16, 32, 64):
        x = x + jnp.where(lane >= d, jnp.roll(x, d, axis=1), 0.0)

    rowtot = x[:, 127:128]
    row = jax.lax.broadcasted_iota(jnp.int32, (_C, 1), 0)
    s = rowtot
    for d in (1, 2, 4, 8, 16, 32, 64, 128, 256):
        s = s + jnp.where(row >= d, jnp.roll(s, d, axis=0), 0.0)
    cum = x + (s - rowtot)

    raw = cum - inc0
    m = raw - jnp.floor(raw * (1.0 / _WT_LEN)) * _WT_LEN
    m_ref[...] = m

    def half_chunk(rr):
        base = pl.multiple_of(rr * 8, 8)
        m8 = m_ref[pl.ds(base, 8), :]
        e8 = e_ref[0, pl.ds(base, 8), :]
        b64 = b64_ref[n, rr]
        woff = pl.multiple_of(b64 * 16, 8)
        a_win = a_ref[pl.ds(woff, 16), :]
        d_win = d_ref[pl.ds(woff, 16), :]
        shift = b64 * 64
        lbase = pl.multiple_of(rr * 1024, 1024)
        rows = []
        for q in range(8):
            mq = m8[q:q + 1, :]
            lowf = jnp.floor(mq)
            alpha = mq - lowf
            rel = ((lowf.astype(jnp.int32) - shift) & (_WT_LEN - 1)) & 127
            relb = jnp.broadcast_to(rel, (16, 128))
            av = jnp.take_along_axis(a_win, relb, axis=1)[:_N_WT]
            dv = jnp.take_along_axis(d_win, relb, axis=1)[:_N_WT]
            val = av + alpha * dv
            attq = att_ref[0, :, pl.ds(lbase + 128 * q, 128)]
            rows.append(jnp.sum(val * attq, axis=0, keepdims=True))
        o_ref[0, pl.ds(base, 8), :] = jnp.concatenate(rows, axis=0) * e8

    def chunk(rr4, carry):
        for t in range(16):
            half_chunk(rr4 * 16 + t)
        return carry

    jax.lax.fori_loop(0, _RR // 16, chunk, 0)


def kernel(pitch, envelope, attention, wavetables):
    n, l = pitch.shape[0], pitch.shape[1]
    pad = _LP - l

    p2 = jnp.pad(pitch[..., 0], ((0, 0), (0, pad))).reshape(n, _C, 128)
    e2 = jnp.pad(envelope[..., 0], ((0, 0), (0, pad))).reshape(n, _C, 128)
    att3 = jnp.pad(attention.transpose(0, 2, 1), ((0, 0), (0, 0), (0, pad)))

    a_tab = wavetables
    d_tab = jnp.roll(wavetables, -1, axis=1) - wavetables

    def windows(t):
        t2 = jnp.concatenate([t, t], axis=1)
        w = jnp.stack([t2[:, 64 * k:64 * k + 128] for k in range(8)])
        return jnp.pad(w, ((0, 0), (0, 16 - _N_WT), (0, 0))).reshape(128, 128)

    a8, d8 = windows(a_tab), windows(d_tab)

    incp = jnp.pad(pitch[..., 0], ((0, 0), (0, pad))) * _INC_SCALE
    blk = incp.reshape(n, _RR, 1024)
    excl = jnp.cumsum(blk.sum(axis=2), axis=1) - blk.sum(axis=2)
    raw_first = excl + blk[:, :, 0] - blk[0, :, 0]
    basef = raw_first - 1.0
    basef = basef - jnp.floor(basef * (1.0 / _WT_LEN)) * _WT_LEN
    b64 = ((basef.astype(jnp.int32) & (_WT_LEN - 1)) >> 6)

    out = pl.pallas_call(
        _synth_body,
        grid_spec=pltpu.PrefetchScalarGridSpec(
            num_scalar_prefetch=1,
            grid=(n,),
            in_specs=[
                pl.BlockSpec((1, _C, 128), lambda i, b: (i, 0, 0)),
                pl.BlockSpec((1, _C, 128), lambda i, b: (0, 0, 0)),
                pl.BlockSpec((1, _C, 128), lambda i, b: (i, 0, 0)),
                pl.BlockSpec((1, _N_WT, _LP), lambda i, b: (i, 0, 0)),
                pl.BlockSpec((128, 128), lambda i, b: (0, 0)),
                pl.BlockSpec((128, 128), lambda i, b: (0, 0)),
            ],
            out_specs=pl.BlockSpec((1, _C, 128), lambda i, b: (i, 0, 0)),
            scratch_shapes=[pltpu.VMEM((_C, 128), jnp.float32)],
        ),
        out_shape=jax.ShapeDtypeStruct((n, _C, 128), jnp.float32),
        compiler_params=pltpu.CompilerParams(
            dimension_semantics=("parallel",)),
    )(b64, p2, p2, e2, att3, a8, d8)

    return out.reshape(n, _LP)[:, :l, None]


# speedup vs baseline: 2.8813x; 1.2196x over previous
import jax
import jax.numpy as jnp
from jax.experimental import pallas as pl
from jax.experimental.pallas import tpu as pltpu

_SR = 16000
_WT_LEN = 512
_N_WT = 10
_INC_SCALE = _WT_LEN / _SR

_L = 64000
_C = _L // 128
_CP = 504
_CS = 512
_RR = _CS // 8


def _synth_body(b64_ref, p_ref, p0_ref, e_ref, att_ref, a_ref, d_ref,
                o_ref, m_ref):
    n = pl.program_id(0)

    inc = p_ref[0] * _INC_SCALE
    inc0 = p0_ref[0] * _INC_SCALE

    lane = jax.lax.broadcasted_iota(jnp.int32, (_C, 128), 1)
    x = inc
    for d in (1, 2, 4, 8, 16, 32, 64):
        x = x + jnp.where(lane >= d, jnp.roll(x, d, axis=1), 0.0)

    rowtot = x[:, 127:128]
    row = jax.lax.broadcasted_iota(jnp.int32, (_C, 1), 0)
    s = rowtot
    for d in (1, 2, 4, 8, 16, 32, 64, 128, 256):
        s = s + jnp.where(row >= d, jnp.roll(s, d, axis=0), 0.0)
    cum = x + (s - rowtot)

    raw = cum - inc0
    m = raw - jnp.floor(raw * (1.0 / _WT_LEN)) * _WT_LEN
    m_ref[pl.ds(0, _C), :] = m

    def half_chunk(rr):
        base = pl.multiple_of(jnp.minimum(rr * 8, _C - 4), 8)
        m8 = m_ref[pl.ds(base, 8), :]
        e8 = e_ref[0, pl.ds(base, 8), :]
        b64 = b64_ref[n, rr]
        woff = pl.multiple_of(b64 * 16, 8)
        a_win = a_ref[pl.ds(woff, 16), :]
        d_win = d_ref[pl.ds(woff, 16), :]
        shift = b64 * 64
        lbase = rr * 1024
        rows = []
        for q in range(8):
            mq = m8[q:q + 1, :]
            lowf = jnp.floor(mq)
            alpha = mq - lowf
            rel = ((lowf.astype(jnp.int32) - shift) & (_WT_LEN - 1)) & 127
            relb = jnp.broadcast_to(rel, (16, 128))
            av = jnp.take_along_axis(a_win, relb, axis=1)[:_N_WT]
            dv = jnp.take_along_axis(d_win, relb, axis=1)[:_N_WT]
            val = av + alpha * dv
            lq = pl.multiple_of(
                jnp.minimum(lbase + 128 * q, _L - 128), 128)
            attq = att_ref[0, :, pl.ds(lq, 128)]
            rows.append(jnp.sum(val * attq, axis=0, keepdims=True))
        o_ref[0, pl.ds(base, 8), :] = jnp.concatenate(rows, axis=0) * e8

    def chunk(i, carry):
        for t in range(16):
            half_chunk(_RR - 1 - (i * 16 + t))
        return carry

    jax.lax.fori_loop(0, _RR // 16, chunk, 0)


def kernel(pitch, envelope, attention, wavetables):
    n, l = pitch.shape[0], pitch.shape[1]

    p2 = pitch.reshape(n, _C, 128)
    e2 = jnp.pad(envelope[..., 0],
                 ((0, 0), (0, _CP * 128 - l))).reshape(n, _CP, 128)
    att3 = attention.transpose(0, 2, 1)

    a_tab = wavetables
    d_tab = jnp.roll(wavetables, -1, axis=1) - wavetables

    def windows(t):
        t2 = jnp.concatenate([t, t], axis=1)
        w = jnp.stack([t2[:, 64 * k:64 * k + 128] for k in range(8)])
        return jnp.pad(w, ((0, 0), (0, 16 - _N_WT), (0, 0))).reshape(128, 128)

    a8, d8 = windows(a_tab), windows(d_tab)

    incp = jnp.pad(pitch[..., 0], ((0, 0), (0, _RR * 1024 - l))) * _INC_SCALE
    blk = incp.reshape(n, _RR, 1024)
    excl = jnp.cumsum(blk.sum(axis=2), axis=1) - blk.sum(axis=2)
    raw_first = excl + blk[:, :, 0] - blk[0, :, 0]
    basef = raw_first - 1.0
    basef = basef - jnp.floor(basef * (1.0 / _WT_LEN)) * _WT_LEN
    b64 = ((basef.astype(jnp.int32) & (_WT_LEN - 1)) >> 6)

    out = pl.pallas_call(
        _synth_body,
        grid_spec=pltpu.PrefetchScalarGridSpec(
            num_scalar_prefetch=1,
            grid=(n,),
            in_specs=[
                pl.BlockSpec((1, _C, 128), lambda i, b: (i, 0, 0)),
                pl.BlockSpec((1, _C, 128), lambda i, b: (0, 0, 0)),
                pl.BlockSpec((1, _CP, 128), lambda i, b: (i, 0, 0)),
                pl.BlockSpec((1, _N_WT, _L), lambda i, b: (i, 0, 0)),
                pl.BlockSpec((128, 128), lambda i, b: (0, 0)),
                pl.BlockSpec((128, 128), lambda i, b: (0, 0)),
            ],
            out_specs=pl.BlockSpec((1, _CP, 128), lambda i, b: (i, 0, 0)),
            scratch_shapes=[pltpu.VMEM((_CS, 128), jnp.float32)],
        ),
        out_shape=jax.ShapeDtypeStruct((n, _CP, 128), jnp.float32),
        compiler_params=pltpu.CompilerParams(
            dimension_semantics=("parallel",)),
    )(b64, p2, p2, e2, att3, a8, d8)

    return out[:, :_C].reshape(n, l, 1)


# speedup vs baseline: 3.2726x; 1.1358x over previous
import jax
import jax.numpy as jnp
from jax.experimental import pallas as pl
from jax.experimental.pallas import tpu as pltpu

_SR = 16000
_WT_LEN = 512
_N_WT = 10
_INC_SCALE = _WT_LEN / _SR

_L = 64000
_C = _L // 128
_CP = 504
_CS = 512
_RR = _CS // 8


def _synth_body(b64_ref, p_ref, p0_ref, e_ref, att_ref, pk_ref,
                o_ref, m_ref):
    n = pl.program_id(0)

    inc = p_ref[0] * _INC_SCALE
    inc0 = p0_ref[0] * _INC_SCALE

    lane = jax.lax.broadcasted_iota(jnp.int32, (_C, 128), 1)
    x = inc
    for d in (1, 2, 4, 8, 16, 32, 64):
        x = x + jnp.where(lane >= d, jnp.roll(x, d, axis=1), 0.0)

    rowtot = x[:, 127:128]
    row = jax.lax.broadcasted_iota(jnp.int32, (_C, 1), 0)
    s = rowtot
    for d in (1, 2, 4, 8, 16, 32, 64, 128, 256):
        s = s + jnp.where(row >= d, jnp.roll(s, d, axis=0), 0.0)
    cum = x + (s - rowtot)

    raw = cum - inc0
    m = raw - jnp.floor(raw * (1.0 / _WT_LEN)) * _WT_LEN
    m_ref[pl.ds(0, _C), :] = m

    def half_chunk(rr):
        base = pl.multiple_of(jnp.minimum(rr * 8, _C - 4), 8)
        m8 = m_ref[pl.ds(base, 8), :]
        e8 = e_ref[0, pl.ds(base, 8), :]
        b64 = b64_ref[n, rr]
        woff = pl.multiple_of(b64 * 16, 8)
        pk_win = pk_ref[pl.ds(woff, 16), :]
        shift = b64 * 64
        lbase = rr * 1024
        rows = []
        for q in range(8):
            mq = m8[q:q + 1, :]
            lowf = jnp.floor(mq)
            alpha = mq - lowf
            rel = ((lowf.astype(jnp.int32) - shift) & (_WT_LEN - 1)) & 127
            relb = jnp.broadcast_to(rel, (16, 128))
            g = jnp.take_along_axis(pk_win, relb, axis=1)[:_N_WT]
            av = jax.lax.bitcast_convert_type(g & -65536, jnp.float32)
            dv = jax.lax.bitcast_convert_type(g << 16, jnp.float32)
            val = av + alpha * dv
            lq = pl.multiple_of(
                jnp.minimum(lbase + 128 * q, _L - 128), 128)
            attq = att_ref[0, :, pl.ds(lq, 128)]
            rows.append(jnp.sum(val * attq, axis=0, keepdims=True))
        o_ref[0, pl.ds(base, 8), :] = jnp.concatenate(rows, axis=0) * e8

    def chunk(i, carry):
        for t in range(16):
            half_chunk(_RR - 1 - (i * 16 + t))
        return carry

    jax.lax.fori_loop(0, _RR // 16, chunk, 0)


def kernel(pitch, envelope, attention, wavetables):
    n, l = pitch.shape[0], pitch.shape[1]

    p2 = pitch.reshape(n, _C, 128)
    e2 = jnp.pad(envelope[..., 0],
                 ((0, 0), (0, _CP * 128 - l))).reshape(n, _CP, 128)
    att3 = attention.transpose(0, 2, 1)

    a_tab = wavetables
    d_tab = jnp.roll(wavetables, -1, axis=1) - wavetables

    def windows(t):
        t2 = jnp.concatenate([t, t], axis=1)
        w = jnp.stack([t2[:, 64 * k:64 * k + 128] for k in range(8)])
        return jnp.pad(w, ((0, 0), (0, 16 - _N_WT), (0, 0))).reshape(128, 128)

    au = jax.lax.bitcast_convert_type(
        windows(a_tab).astype(jnp.bfloat16), jnp.uint16).astype(jnp.uint32)
    du = jax.lax.bitcast_convert_type(
        windows(d_tab).astype(jnp.bfloat16), jnp.uint16).astype(jnp.uint32)
    pk8 = jax.lax.bitcast_convert_type((au << 16) | du, jnp.int32)

    incp = jnp.pad(pitch[..., 0], ((0, 0), (0, _RR * 1024 - l))) * _INC_SCALE
    blk = incp.reshape(n, _RR, 1024)
    excl = jnp.cumsum(blk.sum(axis=2), axis=1) - blk.sum(axis=2)
    raw_first = excl + blk[:, :, 0] - blk[0, :, 0]
    basef = raw_first - 1.0
    basef = basef - jnp.floor(basef * (1.0 / _WT_LEN)) * _WT_LEN
    b64 = ((basef.astype(jnp.int32) & (_WT_LEN - 1)) >> 6)

    out = pl.pallas_call(
        _synth_body,
        grid_spec=pltpu.PrefetchScalarGridSpec(
            num_scalar_prefetch=1,
            grid=(n,),
            in_specs=[
                pl.BlockSpec((1, _C, 128), lambda i, b: (i, 0, 0)),
                pl.BlockSpec((1, _C, 128), lambda i, b: (0, 0, 0)),
                pl.BlockSpec((1, _CP, 128), lambda i, b: (i, 0, 0)),
                pl.BlockSpec((1, _N_WT, _L), lambda i, b: (i, 0, 0)),
                pl.BlockSpec((128, 128), lambda i, b: (0, 0)),
            ],
            out_specs=pl.BlockSpec((1, _CP, 128), lambda i, b: (i, 0, 0)),
            scratch_shapes=[pltpu.VMEM((_CS, 128), jnp.float32)],
        ),
        out_shape=jax.ShapeDtypeStruct((n, _CP, 128), jnp.float32),
        compiler_params=pltpu.CompilerParams(
            dimension_semantics=("parallel",)),
    )(b64, p2, p2, e2, att3, pk8)

    return out[:, :_C].reshape(n, l, 1)
